# baseline (speedup 1.0000x reference)
import functools

import jax
import jax.numpy as jnp
from jax import lax
from jax.experimental import pallas as pl
from jax.experimental.pallas import tpu as pltpu
from jax.experimental.pallas import tpu_sc as plsc

_NC = 2
_NS = 16
_NW = _NC * _NS
_LANES = 16
_K = 4
_TPAD = 64
_GPAD = 56


def _bag_body(tok, d, rw, pidx_hbm, hoff_hbm, w_hbm, out_hbm, pidx_v, hoff_v,
              rows_v, out_v, *sems):
    nvr = d // _LANES
    wid = lax.axis_index("s") * _NC + lax.axis_index("c")
    base = wid * rw
    pltpu.sync_copy(pidx_hbm.at[pl.ds(base, rw)], pidx_v)
    pltpu.sync_copy(hoff_hbm.at[pl.ds(base, rw)], hoff_v)

    lanes = lax.iota(jnp.int32, _LANES)

    def accum(r, buf):
        acc = [jnp.zeros((_LANES,), jnp.float32) for _ in range(nvr)]
        for w in range(0, tok, _LANES):
            off_vec = hoff_v[r, pl.ds(w, _LANES)]
            for u in range(min(_LANES, tok - w)):
                off = jnp.take_along_axis(
                    off_vec, jnp.full((_LANES,), u, jnp.int32), axis=0)
                row_sel = jnp.full((_LANES,), w + u, jnp.int32)
                for j in range(nvr):
                    col = off + (lanes + _LANES * j)
                    acc[j] = acc[j] + plsc.load_gather(
                        rows_v.at[buf], [row_sel, col])
        for j in range(nvr):
            out_v[r, pl.ds(_LANES * j, _LANES)] = acc[j]

    def group_step(g, _):
        descs = []
        for k in range(_K):
            r = g * _K + k
            descs.append(
                pltpu.async_copy(w_hbm.at[pidx_v.at[r]], rows_v.at[k],
                                 sems[k])
            )
        for k in range(_K):
            descs[k].wait()
            accum(g * _K + k, k)
        return _

    lax.fori_loop(0, rw // _K, group_step, 0)
    pltpu.sync_copy(out_v, out_hbm.at[pl.ds(base, rw)])


def _build(batch, tok, vocab, d):
    rw = batch // _NW
    mesh = plsc.VectorSubcoreMesh(core_axis_name="c", subcore_axis_name="s")
    body = functools.partial(_bag_body, tok, d, rw)
    return pl.kernel(
        body,
        out_type=jax.ShapeDtypeStruct((batch, d), jnp.float32),
        mesh=mesh,
        scratch_types=[
            pltpu.VMEM((rw, _GPAD), jnp.int32),
            pltpu.VMEM((rw, _TPAD), jnp.int32),
            pltpu.VMEM((_K, _GPAD, 2 * d), jnp.float32),
            pltpu.VMEM((rw, d), jnp.float32),
        ] + [pltpu.SemaphoreType.DMA] * _K,
        compiler_params=pltpu.CompilerParams(
            use_tc_tiling_on_sc=True, needs_layout_passes=False),
    )


def kernel(contexts, weight):
    batch, tok = contexts.shape
    vocab, d = weight.shape
    ids = contexts.astype(jnp.int32)
    pidx = jnp.pad(ids >> 1, ((0, 0), (0, _GPAD - tok)))
    hoff = jnp.pad((ids & 1) * d, ((0, 0), (0, _TPAD - tok)))
    w2 = jnp.concatenate([weight[0::2, :], weight[1::2, :]], axis=1)
    f = _build(batch, tok, vocab, d)
    return f(pidx, hoff, w2)


# speedup vs baseline: 11.2579x; 11.2579x over previous
import functools

import jax
import jax.numpy as jnp
from jax import lax
from jax.experimental import pallas as pl
from jax.experimental.pallas import tpu as pltpu
from jax.experimental.pallas import tpu_sc as plsc

_NC = 2
_NS = 16
_NW = _NC * _NS
_LANES = 16
_K = 4


def _bag_body(tok, d, rw, ctx_hbm, w_hbm, out_hbm, idx_v, rows_v, out_v,
              *sems):
    nvr = d // _LANES
    wid = lax.axis_index("s") * _NC + lax.axis_index("c")
    base = wid * rw
    pltpu.sync_copy(ctx_hbm.at[pl.ds(base, rw)], idx_v)

    lanes = lax.iota(jnp.int32, _LANES)

    def accum(r, buf):
        def tok_step(t, acc):
            new = []
            for g in range(nvr // 2):
                a, b = plsc.unpack(
                    rows_v[buf, t, pl.ds(2 * _LANES * g, 2 * _LANES)],
                    format=plsc.PackFormat.INTERLEAVED,
                    preferred_element_type=jnp.float32)
                new.append(acc[2 * g] + a)
                new.append(acc[2 * g + 1] + b)
            return tuple(new)
        acc = lax.fori_loop(
            0, tok, tok_step,
            tuple(jnp.zeros((_LANES,), jnp.float32) for _ in range(nvr)),
            unroll=2,
        )
        for g in range(nvr // 2):
            plsc.store_scatter(out_v.at[r], [2 * _LANES * g + 2 * lanes],
                               acc[2 * g])
            plsc.store_scatter(out_v.at[r], [2 * _LANES * g + 2 * lanes + 1],
                               acc[2 * g + 1])

    def group_step(g, _):
        descs = []
        for k in range(_K):
            r = g * _K + k
            descs.append(
                pltpu.async_copy(w_hbm.at[idx_v.at[r]], rows_v.at[k], sems[k])
            )
        for k in range(_K):
            descs[k].wait()
            accum(g * _K + k, k)
        return _

    lax.fori_loop(0, rw // _K, group_step, 0)
    pltpu.sync_copy(out_v, out_hbm.at[pl.ds(base, rw)])


def _build(batch, tok, vocab, d):
    rw = batch // _NW
    mesh = plsc.VectorSubcoreMesh(core_axis_name="c", subcore_axis_name="s")
    body = functools.partial(_bag_body, tok, d, rw)
    return pl.kernel(
        body,
        out_type=jax.ShapeDtypeStruct((batch, d), jnp.float32),
        mesh=mesh,
        scratch_types=[
            pltpu.VMEM((rw, tok), jnp.int32),
            pltpu.VMEM((_K, tok, d), jnp.bfloat16),
            pltpu.VMEM((rw, d), jnp.float32),
        ] + [pltpu.SemaphoreType.DMA] * _K,
        compiler_params=pltpu.CompilerParams(
            use_tc_tiling_on_sc=False, needs_layout_passes=False),
    )


def kernel(contexts, weight):
    batch, tok = contexts.shape
    vocab, d = weight.shape
    f = _build(batch, tok, vocab, d)
    return f(contexts.astype(jnp.int32), weight.astype(jnp.bfloat16))
